# baseline (speedup 1.0000x reference)
import functools

import jax
import jax.numpy as jnp
from jax import lax
from jax.experimental import pallas as pl
from jax.experimental.pallas import tpu as pltpu
from jax.experimental.pallas import tpu_sc as plsc

_NC = 2
_NS = 16
_NW = _NC * _NS
_K = 80
_KP = 128
_DW = 16


def _pad_to(n, m):
    return ((n + m - 1) // m) * m


def _sc_degree(dst_flat, n_pad, e_pt):
    rows_pt = n_pad // _NS
    n_grp = rows_pt // 16
    mesh = plsc.VectorSubcoreMesh(core_axis_name="c", subcore_axis_name="s")

    @functools.partial(
        pl.kernel,
        mesh=mesh,
        out_type=jax.ShapeDtypeStruct((_NC, n_pad, _DW), jnp.float32),
        scratch_types=[
            pltpu.VMEM((e_pt,), jnp.int32),
            pltpu.VMEM((n_pad,), jnp.float32),
            pltpu.VMEM((_NS, rows_pt), jnp.float32),
            pltpu.VMEM((16,), jnp.float32),
            pltpu.VMEM((rows_pt, _DW), jnp.float32),
            pltpu.VMEM_SHARED((_NS, n_pad), jnp.float32),
        ],
        compiler_params=pltpu.CompilerParams(needs_layout_passes=False),
    )
    def k(dst_hbm, degp_hbm, dst_v, hist_v, red_v, acc_v, degw_v, deg_sh):
        cid = lax.axis_index("c")
        sid = lax.axis_index("s")
        w = cid * _NS + sid
        pltpu.sync_copy(dst_hbm.at[w], dst_v)

        def zero_body(i, carry):
            hist_v[pl.ds(i * 16, 16)] = jnp.zeros((16,), jnp.float32)
            return carry

        lax.fori_loop(0, n_pad // 16, zero_body, 0)
        ones = jnp.full((16,), 1.0, jnp.float32)

        def hist_body(i, carry):
            idx = dst_v[pl.ds(i * 16, 16)]
            plsc.addupdate_scatter(hist_v, [idx], ones)
            return carry

        lax.fori_loop(0, e_pt // 16, hist_body, 0)
        pltpu.sync_copy(hist_v, deg_sh.at[sid])
        plsc.subcore_barrier()
        for r in range(_NS):
            pltpu.sync_copy(deg_sh.at[r].at[pl.ds(sid * rows_pt, rows_pt)],
                            red_v.at[r])

        def red_body(j, carry):
            acc = red_v[0, pl.ds(j * 16, 16)]
            for r in range(1, _NS):
                acc = acc + red_v[r, pl.ds(j * 16, 16)]
            acc_v[...] = acc
            for l in range(16):
                degw_v[j * 16 + l, :] = plsc.load_gather(
                    acc_v, [jnp.full((16,), l, jnp.int32)])
            return carry

        lax.fori_loop(0, n_grp, red_body, 0)
        pltpu.sync_copy(degw_v,
                        degp_hbm.at[cid].at[pl.ds(sid * rows_pt, rows_pt)])

    return k(dst_flat)


def _sc_scatter(g_pad, src3, dst3, zeros_row, n_acc, n_chunks):
    rows_pt = n_acc // _NS
    f_out = g_pad.shape[1]
    mesh = plsc.VectorSubcoreMesh(core_axis_name="c", subcore_axis_name="s")

    @functools.partial(
        pl.kernel,
        mesh=mesh,
        out_type=jax.ShapeDtypeStruct((_NC, n_acc, f_out), jnp.float32),
        scratch_types=[
            pltpu.VMEM_SHARED((n_acc, f_out), jnp.float32),
            pltpu.VMEM((n_chunks, _K), jnp.int32),
            pltpu.VMEM((n_chunks, _K), jnp.int32),
            pltpu.VMEM((_K, f_out), jnp.float32),
            pltpu.VMEM((_K, f_out), jnp.float32),
            pltpu.SemaphoreType.DMA,
            pltpu.SemaphoreType.DMA,
        ],
    )
    def k(g_hbm, src_hbm, dst_hbm, zeros_hbm, out_hbm,
          acc_sh, src_v, dst_v, rows0_v, rows1_v, sem0, sem1):
        cid = lax.axis_index("c")
        sid = lax.axis_index("s")
        w = cid * _NS + sid
        pltpu.sync_copy(zeros_hbm, acc_sh.at[pl.ds(sid * rows_pt, rows_pt)])
        pltpu.sync_copy(src_hbm.at[w], src_v)
        pltpu.sync_copy(dst_hbm.at[w], dst_v)
        plsc.subcore_barrier()

        bufs = (rows0_v, sem0), (rows1_v, sem1)
        pltpu.async_copy(g_hbm.at[src_v.at[0]], rows0_v, sem0)
        pltpu.async_copy(g_hbm.at[src_v.at[1]], rows1_v, sem1)

        def body(i, carry):
            for b, (rows_v, sem) in enumerate(bufs):
                c = i * 2 + b
                pltpu.make_async_copy(g_hbm.at[src_v.at[c]], rows_v, sem).wait()
                pltpu.sync_copy(rows_v, acc_sh.at[dst_v.at[c]], add=True)

                @pl.when(c + 2 < n_chunks)
                def _():
                    pltpu.async_copy(g_hbm.at[src_v.at[c + 2]], rows_v, sem)
            return carry

        lax.fori_loop(0, n_chunks // 2, body, 0)
        plsc.subcore_barrier()
        pltpu.sync_copy(
            acc_sh.at[pl.ds(sid * rows_pt, rows_pt)],
            out_hbm.at[cid].at[pl.ds(sid * rows_pt, rows_pt)],
        )

    return k(g_pad, src3, dst3, zeros_row)


def _tc_g(x_pad, W, u2, degp, n_pad):
    f_in = x_pad.shape[1]
    f_out = W.shape[0]
    blk = 512
    grid = n_pad // blk

    def body(x_ref, w_ref, u_ref, deg_ref, g_ref):
        Wm = w_ref[...]
        uv = u_ref[...]
        v = jnp.dot(uv, Wm, preferred_element_type=jnp.float32)
        v = v / (jnp.sqrt(jnp.sum(v * v)) + 1e-12)
        t = lax.dot_general(v, Wm, (((1,), (1,)), ((), ())),
                            preferred_element_type=jnp.float32)
        un = t / (jnp.sqrt(jnp.sum(t * t)) + 1e-12)
        sigma = jnp.sum(un * t)
        Wsn = Wm / sigma
        d = deg_ref[...]
        deg = d[0] + d[1] + 1.0
        dinv = lax.rsqrt(deg[:, :1])
        h = lax.dot_general(x_ref[...], Wsn, (((1,), (1,)), ((), ())),
                            preferred_element_type=jnp.float32)
        g_ref[...] = h * dinv

    return pl.pallas_call(
        body,
        grid=(grid,),
        in_specs=[
            pl.BlockSpec((blk, f_in), lambda i: (i, 0)),
            pl.BlockSpec((f_out, f_in), lambda i: (0, 0)),
            pl.BlockSpec((1, f_out), lambda i: (0, 0)),
            pl.BlockSpec((_NC, blk, _DW), lambda i: (0, i, 0)),
        ],
        out_specs=pl.BlockSpec((blk, f_out), lambda i: (i, 0)),
        out_shape=jax.ShapeDtypeStruct((n_pad, f_out), jnp.float32),
    )(x_pad, W, u2, degp)


def _tc_final(partial, g_pad, degp, b2, alpha11, n_pad):
    f_out = g_pad.shape[1]
    blk = 512
    grid = n_pad // blk

    parts = list(partial)
    np_parts = len(parts)

    def body(*refs):
        p_refs = refs[:np_parts]
        g_ref, deg_ref, b_ref, a_ref, o_ref = refs[np_parts:]
        p = p_refs[0][0] + p_refs[0][1]
        for pr in p_refs[1:]:
            p = p + pr[0] + pr[1]
        d = deg_ref[...]
        deg = d[0] + d[1] + 1.0
        dinv = lax.rsqrt(deg[:, :1])
        out = dinv * (p + g_ref[...]) + b_ref[...]
        alpha = a_ref[0, 0]
        o_ref[...] = jnp.where(out >= 0, out, alpha * out)

    return pl.pallas_call(
        body,
        grid=(grid,),
        in_specs=[pl.BlockSpec((_NC, blk, f_out), lambda i: (0, i, 0))
                  for _ in parts] + [
            pl.BlockSpec((blk, f_out), lambda i: (i, 0)),
            pl.BlockSpec((_NC, blk, _DW), lambda i: (0, i, 0)),
            pl.BlockSpec((1, f_out), lambda i: (0, 0)),
            pl.BlockSpec(memory_space=pltpu.SMEM),
        ],
        out_specs=pl.BlockSpec((blk, f_out), lambda i: (i, 0)),
        out_shape=jax.ShapeDtypeStruct((n_pad, f_out), jnp.float32),
    )(*parts, g_pad, degp, b2, alpha11)


def kernel(x, edge_index, W, b, prelu_alpha, u):
    n, f_in = x.shape
    f_out = W.shape[0]
    e = edge_index.shape[1]
    n_pad = _pad_to(n, 2048)
    e_pad = _pad_to(e, _NW * _K * 4)
    n_chunks = e_pad // (_NW * _K)
    e_pt = e_pad // _NW

    src = edge_index[0]
    dst = edge_index[1]
    if e_pad != e:
        fill_n = jnp.full((e_pad - e,), n, jnp.int32)
        fill_0 = jnp.zeros((e_pad - e,), jnp.int32)
        src_sc = jnp.concatenate([src, fill_n])
        dst_sc = jnp.concatenate([dst, fill_0])
        dst_hist = jnp.concatenate([dst, fill_n])
    else:
        src_sc = src
        dst_sc = dst
        dst_hist = dst
    src3 = src_sc.reshape(_NW, n_chunks, _K)
    dst3 = dst_sc.reshape(_NW, n_chunks, _K)
    dst_flat = dst_hist.reshape(_NW, e_pt)

    zeros_row = jnp.zeros((n_pad // _NS, f_out), jnp.float32)

    degp = _sc_degree(dst_flat, n_pad, e_pt)

    x_pad = jnp.pad(x, ((0, n_pad - n), (0, 0)))
    u2 = u.reshape(1, f_out)
    g_pad = _tc_g(x_pad, W, u2, degp, n_pad)

    ch = n_chunks // 2
    part_a = _sc_scatter(g_pad, src3[:, :ch], dst3[:, :ch], zeros_row,
                         n_pad, ch)
    part_b = _sc_scatter(g_pad, src3[:, ch:], dst3[:, ch:], zeros_row,
                         n_pad, ch)

    b2 = b.reshape(1, f_out)
    alpha11 = prelu_alpha.reshape(1, 1)
    out_pad = _tc_final((part_a, part_b), g_pad, degp, b2, alpha11, n_pad)
    return out_pad[:n]


# speedup vs baseline: 1.0105x; 1.0105x over previous
import functools

import jax
import jax.numpy as jnp
from jax import lax
from jax.experimental import pallas as pl
from jax.experimental.pallas import tpu as pltpu
from jax.experimental.pallas import tpu_sc as plsc

_NC = 2
_NS = 16
_NW = _NC * _NS
_K = 80
_KP = 128
_DW = 16


def _pad_to(n, m):
    return ((n + m - 1) // m) * m


def _sc_degree(dst_flat, n_pad, e_pt):
    rows_pt = n_pad // _NS
    n_grp = rows_pt // 16
    mesh = plsc.VectorSubcoreMesh(core_axis_name="c", subcore_axis_name="s")

    @functools.partial(
        pl.kernel,
        mesh=mesh,
        out_type=jax.ShapeDtypeStruct((_NC, n_pad, _DW), jnp.float32),
        scratch_types=[
            pltpu.VMEM((e_pt,), jnp.int32),
            pltpu.VMEM((n_pad,), jnp.float32),
            pltpu.VMEM((_NS, rows_pt), jnp.float32),
            pltpu.VMEM((16,), jnp.float32),
            pltpu.VMEM((rows_pt, _DW), jnp.float32),
            pltpu.VMEM_SHARED((_NS, n_pad), jnp.float32),
        ],
        compiler_params=pltpu.CompilerParams(needs_layout_passes=False),
    )
    def k(dst_hbm, degp_hbm, dst_v, hist_v, red_v, acc_v, degw_v, deg_sh):
        cid = lax.axis_index("c")
        sid = lax.axis_index("s")
        w = cid * _NS + sid
        pltpu.sync_copy(dst_hbm.at[w], dst_v)

        def zero_body(i, carry):
            hist_v[pl.ds(i * 16, 16)] = jnp.zeros((16,), jnp.float32)
            return carry

        lax.fori_loop(0, n_pad // 16, zero_body, 0)
        ones = jnp.full((16,), 1.0, jnp.float32)

        def hist_body(i, carry):
            idx = dst_v[pl.ds(i * 16, 16)]
            plsc.addupdate_scatter(hist_v, [idx], ones)
            return carry

        lax.fori_loop(0, e_pt // 16, hist_body, 0)
        pltpu.sync_copy(hist_v, deg_sh.at[sid])
        plsc.subcore_barrier()
        for r in range(_NS):
            pltpu.sync_copy(deg_sh.at[r].at[pl.ds(sid * rows_pt, rows_pt)],
                            red_v.at[r])

        def red_body(j, carry):
            acc = red_v[0, pl.ds(j * 16, 16)]
            for r in range(1, _NS):
                acc = acc + red_v[r, pl.ds(j * 16, 16)]
            acc_v[...] = acc
            for l in range(16):
                degw_v[j * 16 + l, :] = plsc.load_gather(
                    acc_v, [jnp.full((16,), l, jnp.int32)])
            return carry

        lax.fori_loop(0, n_grp, red_body, 0)
        pltpu.sync_copy(degw_v,
                        degp_hbm.at[cid].at[pl.ds(sid * rows_pt, rows_pt)])

    return k(dst_flat)


def _sc_scatter(g_pad, src3, dst3, zeros_row, n_acc, n_chunks):
    rows_pt = n_acc // _NS
    f_out = g_pad.shape[1]
    mesh = plsc.VectorSubcoreMesh(core_axis_name="c", subcore_axis_name="s")

    @functools.partial(
        pl.kernel,
        mesh=mesh,
        out_type=jax.ShapeDtypeStruct((_NC, n_acc, f_out), jnp.float32),
        scratch_types=[
            pltpu.VMEM_SHARED((n_acc, f_out), jnp.float32),
            pltpu.VMEM((n_chunks, _K), jnp.int32),
            pltpu.VMEM((n_chunks, _K), jnp.int32),
            pltpu.VMEM((_K, f_out), jnp.float32),
            pltpu.VMEM((_K, f_out), jnp.float32),
            pltpu.SemaphoreType.DMA,
            pltpu.SemaphoreType.DMA,
        ],
    )
    def k(g_hbm, src_hbm, dst_hbm, zeros_hbm, out_hbm,
          acc_sh, src_v, dst_v, rows0_v, rows1_v, sem0, sem1):
        cid = lax.axis_index("c")
        sid = lax.axis_index("s")
        w = cid * _NS + sid
        pltpu.sync_copy(zeros_hbm, acc_sh.at[pl.ds(sid * rows_pt, rows_pt)])
        pltpu.sync_copy(src_hbm.at[w], src_v)
        pltpu.sync_copy(dst_hbm.at[w], dst_v)
        plsc.subcore_barrier()

        bufs = (rows0_v, sem0), (rows1_v, sem1)
        pltpu.async_copy(g_hbm.at[src_v.at[0]], rows0_v, sem0)
        pltpu.async_copy(g_hbm.at[src_v.at[1]], rows1_v, sem1)

        def body(i, carry):
            for b, (rows_v, sem) in enumerate(bufs):
                c = i * 2 + b
                pltpu.make_async_copy(g_hbm.at[src_v.at[c]], rows_v, sem).wait()
                pltpu.sync_copy(rows_v, acc_sh.at[dst_v.at[c]], add=True)

                @pl.when(c + 2 < n_chunks)
                def _():
                    pltpu.async_copy(g_hbm.at[src_v.at[c + 2]], rows_v, sem)
            return carry

        lax.fori_loop(0, n_chunks // 2, body, 0)
        plsc.subcore_barrier()
        pltpu.sync_copy(
            acc_sh.at[pl.ds(sid * rows_pt, rows_pt)],
            out_hbm.at[cid].at[pl.ds(sid * rows_pt, rows_pt)],
        )

    return k(g_pad, src3, dst3, zeros_row)


def _tc_g(x_pad, W, u2, degp, n_pad):
    f_in = x_pad.shape[1]
    f_out = W.shape[0]
    blk = 512
    grid = n_pad // blk

    def body(x_ref, w_ref, u_ref, deg_ref, g_ref):
        Wm = w_ref[...]
        uv = u_ref[...]
        v = jnp.dot(uv, Wm, preferred_element_type=jnp.float32)
        v = v / (jnp.sqrt(jnp.sum(v * v)) + 1e-12)
        t = lax.dot_general(v, Wm, (((1,), (1,)), ((), ())),
                            preferred_element_type=jnp.float32)
        un = t / (jnp.sqrt(jnp.sum(t * t)) + 1e-12)
        sigma = jnp.sum(un * t)
        Wsn = Wm / sigma
        d = deg_ref[...]
        deg = d[0] + d[1] + 1.0
        dinv = lax.rsqrt(deg[:, :1])
        h = lax.dot_general(x_ref[...], Wsn, (((1,), (1,)), ((), ())),
                            preferred_element_type=jnp.float32)
        g_ref[...] = h * dinv

    return pl.pallas_call(
        body,
        grid=(grid,),
        in_specs=[
            pl.BlockSpec((blk, f_in), lambda i: (i, 0)),
            pl.BlockSpec((f_out, f_in), lambda i: (0, 0)),
            pl.BlockSpec((1, f_out), lambda i: (0, 0)),
            pl.BlockSpec((_NC, blk, _DW), lambda i: (0, i, 0)),
        ],
        out_specs=pl.BlockSpec((blk, f_out), lambda i: (i, 0)),
        out_shape=jax.ShapeDtypeStruct((n_pad, f_out), jnp.float32),
    )(x_pad, W, u2, degp)


def _tc_final(partial, g_pad, degp, b2, alpha11, n_pad):
    f_out = g_pad.shape[1]
    blk = 512
    grid = n_pad // blk

    parts = list(partial)
    np_parts = len(parts)

    def body(*refs):
        p_refs = refs[:np_parts]
        g_ref, deg_ref, b_ref, a_ref, o_ref = refs[np_parts:]
        p = p_refs[0][0] + p_refs[0][1]
        for pr in p_refs[1:]:
            p = p + pr[0] + pr[1]
        d = deg_ref[...]
        deg = d[0] + d[1] + 1.0
        dinv = lax.rsqrt(deg[:, :1])
        out = dinv * (p + g_ref[...]) + b_ref[...]
        alpha = a_ref[0, 0]
        o_ref[...] = jnp.where(out >= 0, out, alpha * out)

    return pl.pallas_call(
        body,
        grid=(grid,),
        in_specs=[pl.BlockSpec((_NC, blk, f_out), lambda i: (0, i, 0))
                  for _ in parts] + [
            pl.BlockSpec((blk, f_out), lambda i: (i, 0)),
            pl.BlockSpec((_NC, blk, _DW), lambda i: (0, i, 0)),
            pl.BlockSpec((1, f_out), lambda i: (0, 0)),
            pl.BlockSpec(memory_space=pltpu.SMEM),
        ],
        out_specs=pl.BlockSpec((blk, f_out), lambda i: (i, 0)),
        out_shape=jax.ShapeDtypeStruct((n_pad, f_out), jnp.float32),
    )(*parts, g_pad, degp, b2, alpha11)


def kernel(x, edge_index, W, b, prelu_alpha, u):
    n, f_in = x.shape
    f_out = W.shape[0]
    e = edge_index.shape[1]
    n_pad = _pad_to(n, 2048)
    e_pad = _pad_to(e, _NW * _K * 4)
    n_chunks = e_pad // (_NW * _K)
    e_pt = e_pad // _NW

    src = edge_index[0]
    dst = edge_index[1]
    if e_pad != e:
        fill_n = jnp.full((e_pad - e,), n, jnp.int32)
        pad_ix = jnp.arange(e_pad - e, dtype=jnp.int32)
        src_sc = jnp.concatenate([src, fill_n])
        dst_sc = jnp.concatenate([dst, pad_ix % n])
        dst_hist = jnp.concatenate([dst, n + pad_ix % (n_pad - n)])
    else:
        src_sc = src
        dst_sc = dst
        dst_hist = dst
    src3 = src_sc.reshape(_NW, n_chunks, _K)
    dst3 = dst_sc.reshape(_NW, n_chunks, _K)
    dst_flat = dst_hist.reshape(_NW, e_pt)

    zeros_row = jnp.zeros((n_pad // _NS, f_out), jnp.float32)

    degp = _sc_degree(dst_flat, n_pad, e_pt)

    x_pad = jnp.pad(x, ((0, n_pad - n), (0, 0)))
    u2 = u.reshape(1, f_out)
    g_pad = _tc_g(x_pad, W, u2, degp, n_pad)

    ch = n_chunks // 2
    part_a = _sc_scatter(g_pad, src3[:, :ch], dst3[:, :ch], zeros_row,
                         n_pad, ch)
    part_b = _sc_scatter(g_pad, src3[:, ch:], dst3[:, ch:], zeros_row,
                         n_pad, ch)

    b2 = b.reshape(1, f_out)
    alpha11 = prelu_alpha.reshape(1, 1)
    out_pad = _tc_final((part_a, part_b), g_pad, degp, b2, alpha11, n_pad)
    return out_pad[:n]


# speedup vs baseline: 2.7991x; 2.7699x over previous
import functools

import jax
import jax.numpy as jnp
from jax import lax
from jax.experimental import pallas as pl
from jax.experimental.pallas import tpu as pltpu
from jax.experimental.pallas import tpu_sc as plsc

_NC = 2
_NS = 16
_NW = _NC * _NS
_K = 100
_KP = 128
_DW = 16


def _pad_to(n, m):
    return ((n + m - 1) // m) * m


def _sc_degree(dst_flat, n_pad, e_pt):
    rows_pt = n_pad // _NS
    n_grp = rows_pt // 16
    mesh = plsc.VectorSubcoreMesh(core_axis_name="c", subcore_axis_name="s")

    @functools.partial(
        pl.kernel,
        mesh=mesh,
        out_type=jax.ShapeDtypeStruct((_NC, n_pad, _DW), jnp.float32),
        scratch_types=[
            pltpu.VMEM((e_pt,), jnp.int32),
            pltpu.VMEM((n_pad,), jnp.float32),
            pltpu.VMEM((_NS, rows_pt), jnp.float32),
            pltpu.VMEM((16,), jnp.float32),
            pltpu.VMEM((rows_pt, _DW), jnp.float32),
            pltpu.VMEM_SHARED((_NS, n_pad), jnp.float32),
        ],
        compiler_params=pltpu.CompilerParams(needs_layout_passes=False),
    )
    def k(dst_hbm, degp_hbm, dst_v, hist_v, red_v, acc_v, degw_v, deg_sh):
        cid = lax.axis_index("c")
        sid = lax.axis_index("s")
        w = cid * _NS + sid
        pltpu.sync_copy(dst_hbm.at[w], dst_v)

        def zero_body(i, carry):
            hist_v[pl.ds(i * 16, 16)] = jnp.zeros((16,), jnp.float32)
            return carry

        lax.fori_loop(0, n_pad // 16, zero_body, 0)
        ones = jnp.full((16,), 1.0, jnp.float32)

        def hist_body(i, carry):
            idx = dst_v[pl.ds(i * 16, 16)]
            plsc.addupdate_scatter(hist_v, [idx], ones)
            return carry

        lax.fori_loop(0, e_pt // 16, hist_body, 0)
        pltpu.sync_copy(hist_v, deg_sh.at[sid])
        plsc.subcore_barrier()
        for r in range(_NS):
            pltpu.sync_copy(deg_sh.at[r].at[pl.ds(sid * rows_pt, rows_pt)],
                            red_v.at[r])

        def red_body(j, carry):
            acc = red_v[0, pl.ds(j * 16, 16)]
            for r in range(1, _NS):
                acc = acc + red_v[r, pl.ds(j * 16, 16)]
            acc_v[...] = acc
            for l in range(16):
                degw_v[j * 16 + l, :] = plsc.load_gather(
                    acc_v, [jnp.full((16,), l, jnp.int32)])
            return carry

        lax.fori_loop(0, n_grp, red_body, 0)
        pltpu.sync_copy(degw_v,
                        degp_hbm.at[cid].at[pl.ds(sid * rows_pt, rows_pt)])

    return k(dst_flat)


def _sc_scatter(g_pad, src3, dst3, zeros_row, n_acc, n_chunks):
    rows_pt = n_acc // _NS
    f_out = g_pad.shape[1]
    mesh = plsc.VectorSubcoreMesh(core_axis_name="c", subcore_axis_name="s")

    @functools.partial(
        pl.kernel,
        mesh=mesh,
        out_type=jax.ShapeDtypeStruct((_NC, n_acc, f_out), jnp.float32),
        scratch_types=[
            pltpu.VMEM_SHARED((n_acc, f_out), jnp.float32),
            pltpu.VMEM((n_chunks, _K), jnp.int32),
            pltpu.VMEM((n_chunks, _K), jnp.int32),
            pltpu.VMEM((_K, f_out), jnp.float32),
            pltpu.VMEM((_K, f_out), jnp.float32),
            pltpu.SemaphoreType.DMA,
            pltpu.SemaphoreType.DMA,
        ],
    )
    def k(g_hbm, src_hbm, dst_hbm, zeros_hbm, out_hbm,
          acc_sh, src_v, dst_v, rows0_v, rows1_v, sem0, sem1):
        cid = lax.axis_index("c")
        sid = lax.axis_index("s")
        w = cid * _NS + sid
        pltpu.sync_copy(zeros_hbm, acc_sh.at[pl.ds(sid * rows_pt, rows_pt)])
        pltpu.sync_copy(src_hbm.at[w], src_v)
        pltpu.sync_copy(dst_hbm.at[w], dst_v)
        plsc.subcore_barrier()

        bufs = (rows0_v, sem0), (rows1_v, sem1)
        pltpu.async_copy(g_hbm.at[src_v.at[0]], rows0_v, sem0)
        pltpu.async_copy(g_hbm.at[src_v.at[1]], rows1_v, sem1)

        def body(i, carry):
            for b, (rows_v, sem) in enumerate(bufs):
                c = i * 2 + b
                pltpu.make_async_copy(g_hbm.at[src_v.at[c]], rows_v, sem).wait()
                pltpu.sync_copy(rows_v, acc_sh.at[dst_v.at[c]], add=True)

                @pl.when(c + 2 < n_chunks)
                def _():
                    pltpu.async_copy(g_hbm.at[src_v.at[c + 2]], rows_v, sem)
            return carry

        lax.fori_loop(0, n_chunks // 2, body, 0)
        plsc.subcore_barrier()
        pltpu.sync_copy(
            acc_sh.at[pl.ds(sid * rows_pt, rows_pt)],
            out_hbm.at[cid].at[pl.ds(sid * rows_pt, rows_pt)],
        )

    return k(g_pad, src3, dst3, zeros_row)


def _tc_g(x_pad, W, u2, degp, n_pad):
    f_in = x_pad.shape[1]
    f_out = W.shape[0]
    blk = 512
    grid = n_pad // blk

    def body(x_ref, w_ref, u_ref, deg_ref, g_ref):
        Wm = w_ref[...]
        uv = u_ref[...]
        v = jnp.dot(uv, Wm, preferred_element_type=jnp.float32)
        v = v / (jnp.sqrt(jnp.sum(v * v)) + 1e-12)
        t = lax.dot_general(v, Wm, (((1,), (1,)), ((), ())),
                            preferred_element_type=jnp.float32)
        un = t / (jnp.sqrt(jnp.sum(t * t)) + 1e-12)
        sigma = jnp.sum(un * t)
        Wsn = Wm / sigma
        d = deg_ref[...]
        deg = d[0] + d[1] + 1.0
        dinv = lax.rsqrt(deg[:, :1])
        h = lax.dot_general(x_ref[...], Wsn, (((1,), (1,)), ((), ())),
                            preferred_element_type=jnp.float32)
        g_ref[...] = h * dinv

    return pl.pallas_call(
        body,
        grid=(grid,),
        in_specs=[
            pl.BlockSpec((blk, f_in), lambda i: (i, 0)),
            pl.BlockSpec((f_out, f_in), lambda i: (0, 0)),
            pl.BlockSpec((1, f_out), lambda i: (0, 0)),
            pl.BlockSpec((_NC, blk, _DW), lambda i: (0, i, 0)),
        ],
        out_specs=pl.BlockSpec((blk, f_out), lambda i: (i, 0)),
        out_shape=jax.ShapeDtypeStruct((n_pad, f_out), jnp.float32),
    )(x_pad, W, u2, degp)


def _tc_final(partial, g_pad, degp, b2, alpha11, n_pad):
    f_out = g_pad.shape[1]
    blk = 512
    grid = n_pad // blk

    parts = list(partial)
    np_parts = len(parts)

    def body(*refs):
        p_refs = refs[:np_parts]
        g_ref, deg_ref, b_ref, a_ref, o_ref = refs[np_parts:]
        p = p_refs[0][0] + p_refs[0][1]
        for pr in p_refs[1:]:
            p = p + pr[0] + pr[1]
        d = deg_ref[...]
        deg = d[0] + d[1] + 1.0
        dinv = lax.rsqrt(deg[:, :1])
        out = dinv * (p + g_ref[...]) + b_ref[...]
        alpha = a_ref[0, 0]
        o_ref[...] = jnp.where(out >= 0, out, alpha * out)

    return pl.pallas_call(
        body,
        grid=(grid,),
        in_specs=[pl.BlockSpec((_NC, blk, f_out), lambda i: (0, i, 0))
                  for _ in parts] + [
            pl.BlockSpec((blk, f_out), lambda i: (i, 0)),
            pl.BlockSpec((_NC, blk, _DW), lambda i: (0, i, 0)),
            pl.BlockSpec((1, f_out), lambda i: (0, 0)),
            pl.BlockSpec(memory_space=pltpu.SMEM),
        ],
        out_specs=pl.BlockSpec((blk, f_out), lambda i: (i, 0)),
        out_shape=jax.ShapeDtypeStruct((n_pad, f_out), jnp.float32),
    )(*parts, g_pad, degp, b2, alpha11)


def kernel(x, edge_index, W, b, prelu_alpha, u):
    n, f_in = x.shape
    f_out = W.shape[0]
    e = edge_index.shape[1]
    n_pad = _pad_to(n, 2048)
    e_pad = _pad_to(e, _NW * _K * 4)
    n_chunks = e_pad // (_NW * _K)
    e_pt = e_pad // _NW

    src = edge_index[0]
    dst = edge_index[1]
    if e_pad != e:
        fill_n = jnp.full((e_pad - e,), n, jnp.int32)
        pad_ix = jnp.arange(e_pad - e, dtype=jnp.int32)
        src_sc = jnp.concatenate([src, fill_n])
        dst_sc = jnp.concatenate([dst, pad_ix % n])
        dst_hist = jnp.concatenate([dst, n + pad_ix % (n_pad - n)])
    else:
        src_sc = src
        dst_sc = dst
        dst_hist = dst
    src3 = src_sc.reshape(_NW, n_chunks, _K)
    dst3 = dst_sc.reshape(_NW, n_chunks, _K)
    dst_flat = dst_hist.reshape(_NW, e_pt)

    zeros_row = jnp.zeros((n_pad // _NS, f_out), jnp.float32)

    degp = _sc_degree(dst_flat, n_pad, e_pt)

    x_pad = jnp.pad(x, ((0, n_pad - n), (0, 0)))
    u2 = u.reshape(1, f_out)
    g_pad = _tc_g(x_pad, W, u2, degp, n_pad)

    ch = n_chunks // 2
    part_a = _sc_scatter(g_pad, src3[:, :ch], dst3[:, :ch], zeros_row,
                         n_pad, ch)
    part_b = _sc_scatter(g_pad, src3[:, ch:], dst3[:, ch:], zeros_row,
                         n_pad, ch)

    b2 = b.reshape(1, f_out)
    alpha11 = prelu_alpha.reshape(1, 1)
    out_pad = _tc_final((part_a, part_b), g_pad, degp, b2, alpha11, n_pad)
    return out_pad[:n]


# speedup vs baseline: 2.8629x; 1.0228x over previous
import functools

import jax
import jax.numpy as jnp
from jax import lax
from jax.experimental import pallas as pl
from jax.experimental.pallas import tpu as pltpu
from jax.experimental.pallas import tpu_sc as plsc

_NC = 2
_NS = 16
_NW = _NC * _NS
_K = 125
_KP = 128
_DW = 16


def _pad_to(n, m):
    return ((n + m - 1) // m) * m


def _sc_degree(dst_flat, n_pad, e_pt):
    rows_pt = n_pad // _NS
    n_grp = rows_pt // 16
    mesh = plsc.VectorSubcoreMesh(core_axis_name="c", subcore_axis_name="s")

    @functools.partial(
        pl.kernel,
        mesh=mesh,
        out_type=jax.ShapeDtypeStruct((_NC, n_pad, _DW), jnp.float32),
        scratch_types=[
            pltpu.VMEM((e_pt,), jnp.int32),
            pltpu.VMEM((n_pad,), jnp.float32),
            pltpu.VMEM((_NS, rows_pt), jnp.float32),
            pltpu.VMEM((16,), jnp.float32),
            pltpu.VMEM((rows_pt, _DW), jnp.float32),
            pltpu.VMEM_SHARED((_NS, n_pad), jnp.float32),
        ],
        compiler_params=pltpu.CompilerParams(needs_layout_passes=False),
    )
    def k(dst_hbm, degp_hbm, dst_v, hist_v, red_v, acc_v, degw_v, deg_sh):
        cid = lax.axis_index("c")
        sid = lax.axis_index("s")
        w = cid * _NS + sid
        pltpu.sync_copy(dst_hbm.at[w], dst_v)

        def zero_body(i, carry):
            hist_v[pl.ds(i * 16, 16)] = jnp.zeros((16,), jnp.float32)
            return carry

        lax.fori_loop(0, n_pad // 16, zero_body, 0)
        ones = jnp.full((16,), 1.0, jnp.float32)

        def hist_body(i, carry):
            idx = dst_v[pl.ds(i * 16, 16)]
            plsc.addupdate_scatter(hist_v, [idx], ones)
            return carry

        lax.fori_loop(0, e_pt // 16, hist_body, 0)
        pltpu.sync_copy(hist_v, deg_sh.at[sid])
        plsc.subcore_barrier()
        for r in range(_NS):
            pltpu.sync_copy(deg_sh.at[r].at[pl.ds(sid * rows_pt, rows_pt)],
                            red_v.at[r])

        def red_body(j, carry):
            acc = red_v[0, pl.ds(j * 16, 16)]
            for r in range(1, _NS):
                acc = acc + red_v[r, pl.ds(j * 16, 16)]
            acc_v[...] = acc
            for l in range(16):
                degw_v[j * 16 + l, :] = plsc.load_gather(
                    acc_v, [jnp.full((16,), l, jnp.int32)])
            return carry

        lax.fori_loop(0, n_grp, red_body, 0)
        pltpu.sync_copy(degw_v,
                        degp_hbm.at[cid].at[pl.ds(sid * rows_pt, rows_pt)])

    return k(dst_flat)


def _sc_scatter(g_pad, src3, dst3, zeros_row, n_acc, n_chunks):
    rows_pt = n_acc // _NS
    f_out = g_pad.shape[1]
    mesh = plsc.VectorSubcoreMesh(core_axis_name="c", subcore_axis_name="s")

    @functools.partial(
        pl.kernel,
        mesh=mesh,
        out_type=jax.ShapeDtypeStruct((_NC, n_acc, f_out), jnp.float32),
        scratch_types=[
            pltpu.VMEM_SHARED((n_acc, f_out), jnp.float32),
            pltpu.VMEM((n_chunks, _K), jnp.int32),
            pltpu.VMEM((n_chunks, _K), jnp.int32),
            pltpu.VMEM((_K, f_out), jnp.float32),
            pltpu.VMEM((_K, f_out), jnp.float32),
            pltpu.SemaphoreType.DMA,
            pltpu.SemaphoreType.DMA,
        ],
    )
    def k(g_hbm, src_hbm, dst_hbm, zeros_hbm, out_hbm,
          acc_sh, src_v, dst_v, rows0_v, rows1_v, sem0, sem1):
        cid = lax.axis_index("c")
        sid = lax.axis_index("s")
        w = cid * _NS + sid
        pltpu.sync_copy(zeros_hbm, acc_sh.at[pl.ds(sid * rows_pt, rows_pt)])
        pltpu.sync_copy(src_hbm.at[w], src_v)
        pltpu.sync_copy(dst_hbm.at[w], dst_v)
        plsc.subcore_barrier()

        bufs = (rows0_v, sem0), (rows1_v, sem1)
        pltpu.async_copy(g_hbm.at[src_v.at[0]], rows0_v, sem0)
        pltpu.async_copy(g_hbm.at[src_v.at[1]], rows1_v, sem1)

        def body(i, carry):
            for b, (rows_v, sem) in enumerate(bufs):
                c = i * 2 + b
                pltpu.make_async_copy(g_hbm.at[src_v.at[c]], rows_v, sem).wait()
                pltpu.sync_copy(rows_v, acc_sh.at[dst_v.at[c]], add=True)

                @pl.when(c + 2 < n_chunks)
                def _():
                    pltpu.async_copy(g_hbm.at[src_v.at[c + 2]], rows_v, sem)
            return carry

        lax.fori_loop(0, n_chunks // 2, body, 0)
        plsc.subcore_barrier()
        pltpu.sync_copy(
            acc_sh.at[pl.ds(sid * rows_pt, rows_pt)],
            out_hbm.at[cid].at[pl.ds(sid * rows_pt, rows_pt)],
        )

    return k(g_pad, src3, dst3, zeros_row)


def _tc_g(x_pad, W, u2, degp, n_pad):
    f_in = x_pad.shape[1]
    f_out = W.shape[0]
    blk = 512
    grid = n_pad // blk

    def body(x_ref, w_ref, u_ref, deg_ref, g_ref):
        Wm = w_ref[...]
        uv = u_ref[...]
        v = jnp.dot(uv, Wm, preferred_element_type=jnp.float32)
        v = v / (jnp.sqrt(jnp.sum(v * v)) + 1e-12)
        t = lax.dot_general(v, Wm, (((1,), (1,)), ((), ())),
                            preferred_element_type=jnp.float32)
        un = t / (jnp.sqrt(jnp.sum(t * t)) + 1e-12)
        sigma = jnp.sum(un * t)
        Wsn = Wm / sigma
        d = deg_ref[...]
        deg = d[0] + d[1] + 1.0
        dinv = lax.rsqrt(deg[:, :1])
        h = lax.dot_general(x_ref[...], Wsn, (((1,), (1,)), ((), ())),
                            preferred_element_type=jnp.float32)
        g_ref[...] = h * dinv

    return pl.pallas_call(
        body,
        grid=(grid,),
        in_specs=[
            pl.BlockSpec((blk, f_in), lambda i: (i, 0)),
            pl.BlockSpec((f_out, f_in), lambda i: (0, 0)),
            pl.BlockSpec((1, f_out), lambda i: (0, 0)),
            pl.BlockSpec((_NC, blk, _DW), lambda i: (0, i, 0)),
        ],
        out_specs=pl.BlockSpec((blk, f_out), lambda i: (i, 0)),
        out_shape=jax.ShapeDtypeStruct((n_pad, f_out), jnp.float32),
    )(x_pad, W, u2, degp)


def _tc_final(partial, g_pad, degp, b2, alpha11, n_pad):
    f_out = g_pad.shape[1]
    blk = 512
    grid = n_pad // blk

    parts = list(partial)
    np_parts = len(parts)

    def body(*refs):
        p_refs = refs[:np_parts]
        g_ref, deg_ref, b_ref, a_ref, o_ref = refs[np_parts:]
        p = p_refs[0][0] + p_refs[0][1]
        for pr in p_refs[1:]:
            p = p + pr[0] + pr[1]
        d = deg_ref[...]
        deg = d[0] + d[1] + 1.0
        dinv = lax.rsqrt(deg[:, :1])
        out = dinv * (p + g_ref[...]) + b_ref[...]
        alpha = a_ref[0, 0]
        o_ref[...] = jnp.where(out >= 0, out, alpha * out)

    return pl.pallas_call(
        body,
        grid=(grid,),
        in_specs=[pl.BlockSpec((_NC, blk, f_out), lambda i: (0, i, 0))
                  for _ in parts] + [
            pl.BlockSpec((blk, f_out), lambda i: (i, 0)),
            pl.BlockSpec((_NC, blk, _DW), lambda i: (0, i, 0)),
            pl.BlockSpec((1, f_out), lambda i: (0, 0)),
            pl.BlockSpec(memory_space=pltpu.SMEM),
        ],
        out_specs=pl.BlockSpec((blk, f_out), lambda i: (i, 0)),
        out_shape=jax.ShapeDtypeStruct((n_pad, f_out), jnp.float32),
    )(*parts, g_pad, degp, b2, alpha11)


def kernel(x, edge_index, W, b, prelu_alpha, u):
    n, f_in = x.shape
    f_out = W.shape[0]
    e = edge_index.shape[1]
    n_pad = _pad_to(n, 2048)
    e_pad = _pad_to(e, _NW * _K * 4)
    n_chunks = e_pad // (_NW * _K)
    e_pt = e_pad // _NW

    src = edge_index[0]
    dst = edge_index[1]
    if e_pad != e:
        fill_n = jnp.full((e_pad - e,), n, jnp.int32)
        pad_ix = jnp.arange(e_pad - e, dtype=jnp.int32)
        src_sc = jnp.concatenate([src, fill_n])
        dst_sc = jnp.concatenate([dst, pad_ix % n])
        dst_hist = jnp.concatenate([dst, n + pad_ix % (n_pad - n)])
    else:
        src_sc = src
        dst_sc = dst
        dst_hist = dst
    src3 = src_sc.reshape(_NW, n_chunks, _K)
    dst3 = dst_sc.reshape(_NW, n_chunks, _K)
    dst_flat = dst_hist.reshape(_NW, e_pt)

    zeros_row = jnp.zeros((n_pad // _NS, f_out), jnp.float32)

    degp = _sc_degree(dst_flat, n_pad, e_pt)

    x_pad = jnp.pad(x, ((0, n_pad - n), (0, 0)))
    u2 = u.reshape(1, f_out)
    g_pad = _tc_g(x_pad, W, u2, degp, n_pad)

    ch = n_chunks // 2
    part_a = _sc_scatter(g_pad, src3[:, :ch], dst3[:, :ch], zeros_row,
                         n_pad, ch)
    part_b = _sc_scatter(g_pad, src3[:, ch:], dst3[:, ch:], zeros_row,
                         n_pad, ch)

    b2 = b.reshape(1, f_out)
    alpha11 = prelu_alpha.reshape(1, 1)
    out_pad = _tc_final((part_a, part_b), g_pad, degp, b2, alpha11, n_pad)
    return out_pad[:n]


# speedup vs baseline: 2.8932x; 1.0106x over previous
import functools

import jax
import jax.numpy as jnp
from jax import lax
from jax.experimental import pallas as pl
from jax.experimental.pallas import tpu as pltpu
from jax.experimental.pallas import tpu_sc as plsc

_NC = 2
_NS = 16
_NW = _NC * _NS
_K = 125
_KP = 128
_DW = 16


def _pad_to(n, m):
    return ((n + m - 1) // m) * m


def _sc_degree(dst_flat, n_pad, e_pt):
    rows_pt = n_pad // _NS
    n_grp = rows_pt // 16
    mesh = plsc.VectorSubcoreMesh(core_axis_name="c", subcore_axis_name="s")

    @functools.partial(
        pl.kernel,
        mesh=mesh,
        out_type=jax.ShapeDtypeStruct((_NC, n_pad, _DW), jnp.float32),
        scratch_types=[
            pltpu.VMEM((e_pt,), jnp.int32),
            pltpu.VMEM((n_pad,), jnp.float32),
            pltpu.VMEM((_NS, rows_pt), jnp.float32),
            pltpu.VMEM((16,), jnp.float32),
            pltpu.VMEM((rows_pt, _DW), jnp.float32),
            pltpu.VMEM_SHARED((_NS, n_pad), jnp.float32),
        ],
        compiler_params=pltpu.CompilerParams(needs_layout_passes=False),
    )
    def k(dst_hbm, degp_hbm, dst_v, hist_v, red_v, acc_v, degw_v, deg_sh):
        cid = lax.axis_index("c")
        sid = lax.axis_index("s")
        w = cid * _NS + sid
        pltpu.sync_copy(dst_hbm.at[w], dst_v)

        def zero_body(i, carry):
            hist_v[pl.ds(i * 16, 16)] = jnp.zeros((16,), jnp.float32)
            return carry

        lax.fori_loop(0, n_pad // 16, zero_body, 0)
        ones = jnp.full((16,), 1.0, jnp.float32)

        def hist_body(i, carry):
            idx = dst_v[pl.ds(i * 16, 16)]
            plsc.addupdate_scatter(hist_v, [idx], ones)
            return carry

        lax.fori_loop(0, e_pt // 16, hist_body, 0)
        pltpu.sync_copy(hist_v, deg_sh.at[sid])
        plsc.subcore_barrier()
        for r in range(_NS):
            pltpu.sync_copy(deg_sh.at[r].at[pl.ds(sid * rows_pt, rows_pt)],
                            red_v.at[r])

        def red_body(j, carry):
            acc = red_v[0, pl.ds(j * 16, 16)]
            for r in range(1, _NS):
                acc = acc + red_v[r, pl.ds(j * 16, 16)]
            acc_v[...] = acc
            for l in range(16):
                degw_v[j * 16 + l, :] = plsc.load_gather(
                    acc_v, [jnp.full((16,), l, jnp.int32)])
            return carry

        lax.fori_loop(0, n_grp, red_body, 0)
        pltpu.sync_copy(degw_v,
                        degp_hbm.at[cid].at[pl.ds(sid * rows_pt, rows_pt)])

    return k(dst_flat)


def _sc_scatter(g_pad, src3, dst3, init, n_acc, n_chunks):
    rows_pt = n_acc // _NS
    f_out = g_pad.shape[1]
    per_core_init = init.ndim == 3
    mesh = plsc.VectorSubcoreMesh(core_axis_name="c", subcore_axis_name="s")

    @functools.partial(
        pl.kernel,
        mesh=mesh,
        out_type=jax.ShapeDtypeStruct((_NC, n_acc, f_out), jnp.float32),
        scratch_types=[
            pltpu.VMEM_SHARED((n_acc, f_out), jnp.float32),
            pltpu.VMEM((n_chunks, _K), jnp.int32),
            pltpu.VMEM((n_chunks, _K), jnp.int32),
            pltpu.VMEM((_K, f_out), jnp.float32),
            pltpu.VMEM((_K, f_out), jnp.float32),
            pltpu.SemaphoreType.DMA,
            pltpu.SemaphoreType.DMA,
        ],
    )
    def k(g_hbm, src_hbm, dst_hbm, init_hbm, out_hbm,
          acc_sh, src_v, dst_v, rows0_v, rows1_v, sem0, sem1):
        cid = lax.axis_index("c")
        sid = lax.axis_index("s")
        w = cid * _NS + sid
        if per_core_init:
            pltpu.sync_copy(init_hbm.at[cid].at[pl.ds(sid * rows_pt, rows_pt)],
                            acc_sh.at[pl.ds(sid * rows_pt, rows_pt)])
        else:
            pltpu.sync_copy(init_hbm, acc_sh.at[pl.ds(sid * rows_pt, rows_pt)])
        pltpu.sync_copy(src_hbm.at[w], src_v)
        pltpu.sync_copy(dst_hbm.at[w], dst_v)
        plsc.subcore_barrier()

        bufs = (rows0_v, sem0), (rows1_v, sem1)
        pltpu.async_copy(g_hbm.at[src_v.at[0]], rows0_v, sem0)
        pltpu.async_copy(g_hbm.at[src_v.at[1]], rows1_v, sem1)

        def body(i, carry):
            for b, (rows_v, sem) in enumerate(bufs):
                c = i * 2 + b
                pltpu.make_async_copy(g_hbm.at[src_v.at[c]], rows_v, sem).wait()
                pltpu.sync_copy(rows_v, acc_sh.at[dst_v.at[c]], add=True)

                @pl.when(c + 2 < n_chunks)
                def _():
                    pltpu.async_copy(g_hbm.at[src_v.at[c + 2]], rows_v, sem)
            return carry

        lax.fori_loop(0, n_chunks // 2, body, 0)
        plsc.subcore_barrier()
        pltpu.sync_copy(
            acc_sh.at[pl.ds(sid * rows_pt, rows_pt)],
            out_hbm.at[cid].at[pl.ds(sid * rows_pt, rows_pt)],
        )

    return k(g_pad, src3, dst3, init)


def _tc_g(x_pad, W, u2, degp, n_pad):
    f_in = x_pad.shape[1]
    f_out = W.shape[0]
    blk = 512
    grid = n_pad // blk

    def body(x_ref, w_ref, u_ref, deg_ref, g_ref):
        Wm = w_ref[...]
        uv = u_ref[...]
        v = jnp.dot(uv, Wm, preferred_element_type=jnp.float32)
        v = v / (jnp.sqrt(jnp.sum(v * v)) + 1e-12)
        t = lax.dot_general(v, Wm, (((1,), (1,)), ((), ())),
                            preferred_element_type=jnp.float32)
        un = t / (jnp.sqrt(jnp.sum(t * t)) + 1e-12)
        sigma = jnp.sum(un * t)
        Wsn = Wm / sigma
        d = deg_ref[...]
        deg = d[0] + d[1] + 1.0
        dinv = lax.rsqrt(deg[:, :1])
        h = lax.dot_general(x_ref[...], Wsn, (((1,), (1,)), ((), ())),
                            preferred_element_type=jnp.float32)
        g_ref[...] = h * dinv

    return pl.pallas_call(
        body,
        grid=(grid,),
        in_specs=[
            pl.BlockSpec((blk, f_in), lambda i: (i, 0)),
            pl.BlockSpec((f_out, f_in), lambda i: (0, 0)),
            pl.BlockSpec((1, f_out), lambda i: (0, 0)),
            pl.BlockSpec((_NC, blk, _DW), lambda i: (0, i, 0)),
        ],
        out_specs=pl.BlockSpec((blk, f_out), lambda i: (i, 0)),
        out_shape=jax.ShapeDtypeStruct((n_pad, f_out), jnp.float32),
    )(x_pad, W, u2, degp)


def _tc_final(partial, g_pad, degp, b2, alpha11, n_pad):
    f_out = g_pad.shape[1]
    blk = 512
    grid = n_pad // blk

    parts = list(partial)
    np_parts = len(parts)

    def body(*refs):
        p_refs = refs[:np_parts]
        g_ref, deg_ref, b_ref, a_ref, o_ref = refs[np_parts:]
        p = p_refs[0][0] + p_refs[0][1]
        for pr in p_refs[1:]:
            p = p + pr[0] + pr[1]
        d = deg_ref[...]
        deg = d[0] + d[1] + 1.0
        dinv = lax.rsqrt(deg[:, :1])
        out = dinv * (p + g_ref[...]) + b_ref[...]
        alpha = a_ref[0, 0]
        o_ref[...] = jnp.where(out >= 0, out, alpha * out)

    return pl.pallas_call(
        body,
        grid=(grid,),
        in_specs=[pl.BlockSpec((_NC, blk, f_out), lambda i: (0, i, 0))
                  for _ in parts] + [
            pl.BlockSpec((blk, f_out), lambda i: (i, 0)),
            pl.BlockSpec((_NC, blk, _DW), lambda i: (0, i, 0)),
            pl.BlockSpec((1, f_out), lambda i: (0, 0)),
            pl.BlockSpec(memory_space=pltpu.SMEM),
        ],
        out_specs=pl.BlockSpec((blk, f_out), lambda i: (i, 0)),
        out_shape=jax.ShapeDtypeStruct((n_pad, f_out), jnp.float32),
    )(*parts, g_pad, degp, b2, alpha11)


def kernel(x, edge_index, W, b, prelu_alpha, u):
    n, f_in = x.shape
    f_out = W.shape[0]
    e = edge_index.shape[1]
    n_pad = _pad_to(n, 2048)
    e_pad = _pad_to(e, _NW * _K * 4)
    n_chunks = e_pad // (_NW * _K)
    e_pt = e_pad // _NW

    src = edge_index[0]
    dst = edge_index[1]
    if e_pad != e:
        fill_n = jnp.full((e_pad - e,), n, jnp.int32)
        pad_ix = jnp.arange(e_pad - e, dtype=jnp.int32)
        src_sc = jnp.concatenate([src, fill_n])
        dst_sc = jnp.concatenate([dst, pad_ix % n])
        dst_hist = jnp.concatenate([dst, n + pad_ix % (n_pad - n)])
    else:
        src_sc = src
        dst_sc = dst
        dst_hist = dst
    src3 = src_sc.reshape(_NW, n_chunks, _K)
    dst3 = dst_sc.reshape(_NW, n_chunks, _K)
    dst_flat = dst_hist.reshape(_NW, e_pt)

    zeros_row = jnp.zeros((n_pad // _NS, f_out), jnp.float32)

    degp = _sc_degree(dst_flat, n_pad, e_pt)

    x_pad = jnp.pad(x, ((0, n_pad - n), (0, 0)))
    u2 = u.reshape(1, f_out)
    g_pad = _tc_g(x_pad, W, u2, degp, n_pad)

    ch = n_chunks // 2
    part_a = _sc_scatter(g_pad, src3[:, :ch], dst3[:, :ch], zeros_row,
                         n_pad, ch)
    part_b = _sc_scatter(g_pad, src3[:, ch:], dst3[:, ch:], part_a,
                         n_pad, ch)

    b2 = b.reshape(1, f_out)
    alpha11 = prelu_alpha.reshape(1, 1)
    out_pad = _tc_final((part_b,), g_pad, degp, b2, alpha11, n_pad)
    return out_pad[:n]


# speedup vs baseline: 2.8960x; 1.0010x over previous
import functools

import jax
import jax.numpy as jnp
from jax import lax
from jax.experimental import pallas as pl
from jax.experimental.pallas import tpu as pltpu
from jax.experimental.pallas import tpu_sc as plsc

_NC = 2
_NS = 16
_NW = _NC * _NS
_K = 125
_DW = 16


def _pad_to(n, m):
    return ((n + m - 1) // m) * m


def _sc_degree(dst_flat, n_pad, e_pt):
    rows_pt = n_pad // _NS
    n_grp = rows_pt // 16
    mesh = plsc.VectorSubcoreMesh(core_axis_name="c", subcore_axis_name="s")

    @functools.partial(
        pl.kernel,
        mesh=mesh,
        out_type=jax.ShapeDtypeStruct((_NC, n_pad, _DW), jnp.float32),
        scratch_types=[
            pltpu.VMEM((e_pt,), jnp.int32),
            pltpu.VMEM((n_pad,), jnp.float32),
            pltpu.VMEM((_NS, rows_pt), jnp.float32),
            pltpu.VMEM((16,), jnp.float32),
            pltpu.VMEM((rows_pt, _DW), jnp.float32),
            pltpu.VMEM_SHARED((_NS, n_pad), jnp.float32),
        ],
        compiler_params=pltpu.CompilerParams(needs_layout_passes=False),
    )
    def k(dst_hbm, degp_hbm, dst_v, hist_v, red_v, acc_v, degw_v, deg_sh):
        cid = lax.axis_index("c")
        sid = lax.axis_index("s")
        w = cid * _NS + sid
        pltpu.sync_copy(dst_hbm.at[w], dst_v)

        def zero_body(i, carry):
            hist_v[pl.ds(i * 16, 16)] = jnp.zeros((16,), jnp.float32)
            return carry

        lax.fori_loop(0, n_pad // 16, zero_body, 0)
        ones = jnp.full((16,), 1.0, jnp.float32)

        def hist_body(i, carry):
            idx = dst_v[pl.ds(i * 16, 16)]
            plsc.addupdate_scatter(hist_v, [idx], ones)
            return carry

        lax.fori_loop(0, e_pt // 16, hist_body, 0)
        pltpu.sync_copy(hist_v, deg_sh.at[sid])
        plsc.subcore_barrier()
        for r in range(_NS):
            pltpu.sync_copy(deg_sh.at[r].at[pl.ds(sid * rows_pt, rows_pt)],
                            red_v.at[r])

        def red_body(j, carry):
            acc = red_v[0, pl.ds(j * 16, 16)]
            for r in range(1, _NS):
                acc = acc + red_v[r, pl.ds(j * 16, 16)]
            acc_v[...] = acc
            for l in range(16):
                degw_v[j * 16 + l, :] = plsc.load_gather(
                    acc_v, [jnp.full((16,), l, jnp.int32)])
            return carry

        lax.fori_loop(0, n_grp, red_body, 0)
        pltpu.sync_copy(degw_v,
                        degp_hbm.at[cid].at[pl.ds(sid * rows_pt, rows_pt)])

    return k(dst_flat)


def _sc_scatter(g_pad, src3, dst3, init, n_acc, n_chunks):
    rows_pt = n_acc // _NS
    f_out = g_pad.shape[1]
    per_core_init = init.ndim == 3
    mesh = plsc.VectorSubcoreMesh(core_axis_name="c", subcore_axis_name="s")

    @functools.partial(
        pl.kernel,
        mesh=mesh,
        out_type=jax.ShapeDtypeStruct((_NC, n_acc, f_out), jnp.float32),
        scratch_types=[
            pltpu.VMEM_SHARED((n_acc, f_out), jnp.float32),
            pltpu.VMEM((n_chunks, _K), jnp.int32),
            pltpu.VMEM((n_chunks, _K), jnp.int32),
            pltpu.VMEM((_K, f_out), jnp.float32),
            pltpu.VMEM((_K, f_out), jnp.float32),
            pltpu.SemaphoreType.DMA,
            pltpu.SemaphoreType.DMA,
        ],
    )
    def k(g_hbm, src_hbm, dst_hbm, init_hbm, out_hbm,
          acc_sh, src_v, dst_v, rows0_v, rows1_v, sem0, sem1):
        cid = lax.axis_index("c")
        sid = lax.axis_index("s")
        w = cid * _NS + sid
        if per_core_init:
            pltpu.sync_copy(init_hbm.at[cid].at[pl.ds(sid * rows_pt, rows_pt)],
                            acc_sh.at[pl.ds(sid * rows_pt, rows_pt)])
        else:
            pltpu.sync_copy(init_hbm, acc_sh.at[pl.ds(sid * rows_pt, rows_pt)])
        pltpu.sync_copy(src_hbm.at[w], src_v)
        pltpu.sync_copy(dst_hbm.at[w], dst_v)
        plsc.subcore_barrier()

        bufs = (rows0_v, sem0), (rows1_v, sem1)
        pltpu.async_copy(g_hbm.at[src_v.at[0]], rows0_v, sem0)
        pltpu.async_copy(g_hbm.at[src_v.at[1]], rows1_v, sem1)

        def body(i, carry):
            for b, (rows_v, sem) in enumerate(bufs):
                c = i * 2 + b
                pltpu.make_async_copy(g_hbm.at[src_v.at[c]], rows_v, sem).wait()
                pltpu.sync_copy(rows_v, acc_sh.at[dst_v.at[c]], add=True)

                @pl.when(c + 2 < n_chunks)
                def _():
                    pltpu.async_copy(g_hbm.at[src_v.at[c + 2]], rows_v, sem)
            return carry

        lax.fori_loop(0, n_chunks // 2, body, 0)
        plsc.subcore_barrier()
        pltpu.sync_copy(
            acc_sh.at[pl.ds(sid * rows_pt, rows_pt)],
            out_hbm.at[cid].at[pl.ds(sid * rows_pt, rows_pt)],
        )

    return k(g_pad, src3, dst3, init)


def _tc_g(x_pad, W, u2, degp, n_pad):
    f_in = x_pad.shape[1]
    f_out = W.shape[0]
    blk = 512
    grid = n_pad // blk

    def body(x_ref, w_ref, u_ref, deg_ref, g_ref):
        Wm = w_ref[...]
        uv = u_ref[...]
        v = jnp.dot(uv, Wm, preferred_element_type=jnp.float32)
        v = v / (jnp.sqrt(jnp.sum(v * v)) + 1e-12)
        t = lax.dot_general(v, Wm, (((1,), (1,)), ((), ())),
                            preferred_element_type=jnp.float32)
        un = t / (jnp.sqrt(jnp.sum(t * t)) + 1e-12)
        sigma = jnp.sum(un * t)
        Wsn = Wm / sigma
        d = deg_ref[...]
        deg = d[0] + d[1] + 1.0
        dinv = lax.rsqrt(deg[:, :1])
        h = lax.dot_general(x_ref[...], Wsn, (((1,), (1,)), ((), ())),
                            preferred_element_type=jnp.float32)
        g_ref[...] = h * dinv

    return pl.pallas_call(
        body,
        grid=(grid,),
        in_specs=[
            pl.BlockSpec((blk, f_in), lambda i: (i, 0)),
            pl.BlockSpec((f_out, f_in), lambda i: (0, 0)),
            pl.BlockSpec((1, f_out), lambda i: (0, 0)),
            pl.BlockSpec((_NC, blk, _DW), lambda i: (0, i, 0)),
        ],
        out_specs=pl.BlockSpec((blk, f_out), lambda i: (i, 0)),
        out_shape=jax.ShapeDtypeStruct((n_pad, f_out), jnp.float32),
    )(x_pad, W, u2, degp)


def _tc_final(partial, g_pad, degp, b2, alpha11, n_pad):
    f_out = g_pad.shape[1]
    blk = 512
    grid = n_pad // blk

    parts = list(partial)
    np_parts = len(parts)

    def body(*refs):
        p_refs = refs[:np_parts]
        g_ref, deg_ref, b_ref, a_ref, o_ref = refs[np_parts:]
        p = p_refs[0][0] + p_refs[0][1]
        for pr in p_refs[1:]:
            p = p + pr[0] + pr[1]
        d = deg_ref[...]
        deg = d[0] + d[1] + 1.0
        dinv = lax.rsqrt(deg[:, :1])
        out = dinv * (p + g_ref[...]) + b_ref[...]
        alpha = a_ref[0, 0]
        o_ref[...] = jnp.where(out >= 0, out, alpha * out)

    return pl.pallas_call(
        body,
        grid=(grid,),
        in_specs=[pl.BlockSpec((_NC, blk, f_out), lambda i: (0, i, 0))
                  for _ in parts] + [
            pl.BlockSpec((blk, f_out), lambda i: (i, 0)),
            pl.BlockSpec((_NC, blk, _DW), lambda i: (0, i, 0)),
            pl.BlockSpec((1, f_out), lambda i: (0, 0)),
            pl.BlockSpec(memory_space=pltpu.SMEM),
        ],
        out_specs=pl.BlockSpec((blk, f_out), lambda i: (i, 0)),
        out_shape=jax.ShapeDtypeStruct((n_pad, f_out), jnp.float32),
    )(*parts, g_pad, degp, b2, alpha11)


def kernel(x, edge_index, W, b, prelu_alpha, u):
    n, f_in = x.shape
    f_out = W.shape[0]
    e = edge_index.shape[1]
    n_pad = _pad_to(n, 2048)
    e_pad = _pad_to(e, _NW * _K * 4)
    n_chunks = e_pad // (_NW * _K)
    e_pt = e_pad // _NW

    src = edge_index[0]
    dst = edge_index[1]
    if e_pad != e:
        fill_n = jnp.full((e_pad - e,), n, jnp.int32)
        pad_ix = jnp.arange(e_pad - e, dtype=jnp.int32)
        src_sc = jnp.concatenate([src, fill_n])
        dst_sc = jnp.concatenate([dst, pad_ix % n])
        dst_hist = jnp.concatenate([dst, n + pad_ix % (n_pad - n)])
    else:
        src_sc = src
        dst_sc = dst
        dst_hist = dst
    src3 = src_sc.reshape(_NW, n_chunks, _K)
    dst3 = dst_sc.reshape(_NW, n_chunks, _K)
    dst_flat = dst_hist.reshape(_NW, e_pt)

    zeros_row = jnp.zeros((n_pad // _NS, f_out), jnp.float32)

    degp = _sc_degree(dst_flat, n_pad, e_pt)

    x_pad = jnp.pad(x, ((0, n_pad - n), (0, 0)))
    u2 = u.reshape(1, f_out)
    g_pad = _tc_g(x_pad, W, u2, degp, n_pad)

    ch = n_chunks // 2
    part_a = _sc_scatter(g_pad, src3[:, :ch], dst3[:, :ch], zeros_row,
                         n_pad, ch)
    part_b = _sc_scatter(g_pad, src3[:, ch:], dst3[:, ch:], part_a,
                         n_pad, ch)

    b2 = b.reshape(1, f_out)
    alpha11 = prelu_alpha.reshape(1, 1)
    out_pad = _tc_final((part_b,), g_pad, degp, b2, alpha11, n_pad)
    return out_pad[:n]
